# baseline (speedup 1.0000x reference)
import functools

import jax
import jax.numpy as jnp
from jax import lax
from jax.experimental import pallas as pl
from jax.experimental.pallas import tpu as pltpu
from jax.experimental.pallas import tpu_sc as plsc

NUM_EMB = 1000000
D = 64
DP = 128
B = 4096
S = 200
TOTAL = B * S
NC = 2
NS = 16
NW = NC * NS
PER_W = TOTAL // NW
CHUNK = 128
N_CHUNK = PER_W // CHUNK
NBUF = 4
N_ROUND = N_CHUNK // NBUF

_mesh = plsc.VectorSubcoreMesh(core_axis_name="c", subcore_axis_name="s")


@functools.partial(
    pl.kernel,
    out_type=jax.ShapeDtypeStruct((TOTAL, DP), jnp.float32),
    mesh=_mesh,
    compiler_params=pltpu.CompilerParams(use_tc_tiling_on_sc=False),
    scratch_types=[
        pltpu.VMEM((N_CHUNK, CHUNK), jnp.int32),
        pltpu.VMEM((NBUF, CHUNK, DP), jnp.float32),
        pltpu.SemaphoreType.DMA((NBUF,)),
        pltpu.SemaphoreType.DMA((NBUF,)),
    ],
)
def _embed_sc(table_hbm, idx_hbm, out_hbm, idx_v, rows_v, gsem, ssem):
    wid = lax.axis_index("s") * NC + lax.axis_index("c")
    pltpu.sync_copy(idx_hbm.at[pl.ds(wid * N_CHUNK, N_CHUNK)], idx_v)
    base = wid * PER_W

    def fire_gather(g, b):
        pltpu.async_copy(table_hbm.at[idx_v.at[g]], rows_v.at[b], gsem.at[b])

    def wait_gather(b):
        pltpu.make_async_copy(
            table_hbm.at[idx_v.at[0]], rows_v.at[b], gsem.at[b]).wait()

    def fire_store(g, b):
        pltpu.async_copy(
            rows_v.at[b], out_hbm.at[pl.ds(base + g * CHUNK, CHUNK)],
            ssem.at[b])

    def wait_store(b):
        pltpu.make_async_copy(
            rows_v.at[b], out_hbm.at[pl.ds(base, CHUNK)], ssem.at[b]).wait()

    for b in range(NBUF):
        fire_gather(b, b)

    @pl.loop(0, N_ROUND - 1)
    def _round(r):
        g0 = r * NBUF
        for b in range(NBUF):
            wait_gather(b)
            fire_store(g0 + b, b)
        for b in range(NBUF):
            wait_store(b)
            fire_gather(g0 + NBUF + b, b)

    g0 = (N_ROUND - 1) * NBUF
    for b in range(NBUF):
        wait_gather(b)
        fire_store(g0 + b, b)
    for b in range(NBUF):
        wait_store(b)


BW = 512
TP_GRID = (NUM_EMB + BW - 1) // BW


def _tp_body(x_ref, o_ref):
    o_ref[...] = jnp.concatenate(
        [x_ref[...].T, jnp.zeros((BW, DP - D), jnp.float32)], axis=1)


_pad_transpose_tc = pl.pallas_call(
    _tp_body,
    grid=(TP_GRID,),
    in_specs=[pl.BlockSpec((D, BW), lambda i: (0, i))],
    out_specs=pl.BlockSpec((BW, DP), lambda i: (i, 0)),
    out_shape=jax.ShapeDtypeStruct((NUM_EMB, DP), jnp.float32),
)


def kernel(token_ids, embeddings):
    padded = _pad_transpose_tc(embeddings.T)
    flat = token_ids.reshape(NW * N_CHUNK, CHUNK)
    res = _embed_sc(padded, flat)
    return res[:, :D].reshape(B, S, D)


# speedup vs baseline: 1.8264x; 1.8264x over previous
import functools

import jax
import jax.numpy as jnp
from jax import lax
from jax.experimental import pallas as pl
from jax.experimental.pallas import tpu as pltpu
from jax.experimental.pallas import tpu_sc as plsc

NUM_EMB = 1000000
D = 64
DP = 128
B = 4096
S = 200
TOTAL = B * S
NC = 2
NS = 16
NW = NC * NS
PER_W = TOTAL // NW
CHUNK = 128
N_CHUNK = PER_W // CHUNK
NBUF = 4
N_ROUND = N_CHUNK // NBUF

_mesh = plsc.VectorSubcoreMesh(core_axis_name="c", subcore_axis_name="s")


@functools.partial(
    pl.kernel,
    out_type=jax.ShapeDtypeStruct((TOTAL, DP), jnp.float32),
    mesh=_mesh,
    compiler_params=pltpu.CompilerParams(use_tc_tiling_on_sc=False),
    scratch_types=[
        pltpu.VMEM((N_CHUNK, CHUNK), jnp.int32),
        pltpu.VMEM((NBUF, CHUNK, D), jnp.float32),
        pltpu.SemaphoreType.DMA((NBUF,)),
        pltpu.SemaphoreType.DMA((NBUF,)),
    ],
)
def _embed_sc(table_hbm, idx_hbm, out_hbm, idx_v, rows_v, gsem, ssem):
    wid = lax.axis_index("s") * NC + lax.axis_index("c")
    pltpu.sync_copy(idx_hbm.at[pl.ds(wid * N_CHUNK, N_CHUNK)], idx_v)
    base = wid * PER_W

    def fire_gather(g, b):
        pltpu.async_copy(table_hbm.at[idx_v.at[g]], rows_v.at[b], gsem.at[b])

    def wait_gather(b):
        pltpu.make_async_copy(
            table_hbm.at[idx_v.at[0]], rows_v.at[b], gsem.at[b]).wait()

    def fire_store(g, b):
        pltpu.async_copy(
            rows_v.at[b],
            out_hbm.at[pl.ds(base + g * CHUNK, CHUNK), pl.ds(0, D)],
            ssem.at[b])

    def wait_store(b):
        pltpu.make_async_copy(
            rows_v.at[b], out_hbm.at[pl.ds(base, CHUNK), pl.ds(0, D)],
            ssem.at[b]).wait()

    for b in range(NBUF):
        fire_gather(b, b)

    @pl.loop(0, N_ROUND - 1)
    def _round(r):
        g0 = r * NBUF
        for b in range(NBUF):
            wait_gather(b)
            fire_store(g0 + b, b)
        for b in range(NBUF):
            wait_store(b)
            fire_gather(g0 + NBUF + b, b)

    g0 = (N_ROUND - 1) * NBUF
    for b in range(NBUF):
        wait_gather(b)
        fire_store(g0 + b, b)
    for b in range(NBUF):
        wait_store(b)


def kernel(token_ids, embeddings):
    flat = token_ids.reshape(NW * N_CHUNK, CHUNK)
    res = _embed_sc(embeddings, flat)
    return res[:, :D].reshape(B, S, D)


# speedup vs baseline: 1.8270x; 1.0003x over previous
import functools

import jax
import jax.numpy as jnp
from jax import lax
from jax.experimental import pallas as pl
from jax.experimental.pallas import tpu as pltpu
from jax.experimental.pallas import tpu_sc as plsc

NUM_EMB = 1000000
D = 64
DP = 128
B = 4096
S = 200
TOTAL = B * S
NC = 2
NS = 16
NW = NC * NS
PER_W = TOTAL // NW
CHUNK = 128
N_CHUNK = PER_W // CHUNK
NBUF = 8
N_ROUND = N_CHUNK // NBUF

_mesh = plsc.VectorSubcoreMesh(core_axis_name="c", subcore_axis_name="s")


@functools.partial(
    pl.kernel,
    out_type=jax.ShapeDtypeStruct((TOTAL, DP), jnp.float32),
    mesh=_mesh,
    compiler_params=pltpu.CompilerParams(use_tc_tiling_on_sc=False),
    scratch_types=[
        pltpu.VMEM((N_CHUNK, CHUNK), jnp.int32),
        pltpu.VMEM((NBUF, CHUNK, D), jnp.float32),
        pltpu.SemaphoreType.DMA((NBUF,)),
        pltpu.SemaphoreType.DMA((NBUF,)),
    ],
)
def _embed_sc(table_hbm, idx_hbm, out_hbm, idx_v, rows_v, gsem, ssem):
    wid = lax.axis_index("s") * NC + lax.axis_index("c")
    pltpu.sync_copy(idx_hbm.at[pl.ds(wid * N_CHUNK, N_CHUNK)], idx_v)
    base = wid * PER_W

    def fire_gather(g, b):
        pltpu.async_copy(table_hbm.at[idx_v.at[g]], rows_v.at[b], gsem.at[b])

    def wait_gather(b):
        pltpu.make_async_copy(
            table_hbm.at[idx_v.at[0]], rows_v.at[b], gsem.at[b]).wait()

    def fire_store(g, b):
        pltpu.async_copy(
            rows_v.at[b],
            out_hbm.at[pl.ds(base + g * CHUNK, CHUNK), pl.ds(0, D)],
            ssem.at[b])

    def wait_store(b):
        pltpu.make_async_copy(
            rows_v.at[b], out_hbm.at[pl.ds(base, CHUNK), pl.ds(0, D)],
            ssem.at[b]).wait()

    for b in range(NBUF):
        fire_gather(b, b)

    @pl.loop(0, N_ROUND - 1)
    def _round(r):
        g0 = r * NBUF
        for b in range(NBUF):
            wait_gather(b)
            fire_store(g0 + b, b)
        for b in range(NBUF):
            wait_store(b)
            fire_gather(g0 + NBUF + b, b)

    g0 = (N_ROUND - 1) * NBUF
    for b in range(NBUF):
        wait_gather(b)
        fire_store(g0 + b, b)
    for b in range(NBUF):
        wait_store(b)


def kernel(token_ids, embeddings):
    flat = token_ids.reshape(NW * N_CHUNK, CHUNK)
    res = _embed_sc(embeddings, flat)
    return res[:, :D].reshape(B, S, D)
